# baseline (speedup 1.0000x reference)
import functools

import jax
import jax.numpy as jnp
from jax import lax
from jax.experimental import pallas as pl
from jax.experimental.pallas import tpu as pltpu
from jax.experimental.pallas import tpu_sc as plsc

NC = 2
NS = 16
NW = NC * NS

CHUNK = 512


@functools.cache
def _build_gather(B, V, D):
    assert B % NW == 0
    bpw = B // NW
    assert bpw % CHUNK == 0
    n_chunks = bpw // CHUNK

    mesh = plsc.VectorSubcoreMesh(core_axis_name="c", subcore_axis_name="s")

    @functools.partial(
        pl.kernel,
        out_type=jax.ShapeDtypeStruct((B, D), jnp.float32),
        mesh=mesh,
        compiler_params=pltpu.CompilerParams(use_tc_tiling_on_sc=False),
        scratch_types=[
            pltpu.VMEM((bpw,), jnp.int32),
            pltpu.VMEM((CHUNK, D), jnp.float32),
            pltpu.VMEM((CHUNK, D), jnp.float32),
            pltpu.SemaphoreType.DMA,
            pltpu.SemaphoreType.DMA,
            pltpu.SemaphoreType.DMA,
            pltpu.SemaphoreType.DMA,
        ],
    )
    def gather_kernel(table_hbm, idx_hbm, out_hbm, idx_v, rows0, rows1,
                      gsem0, gsem1, osem0, osem1):
        wid = lax.axis_index("s") * NC + lax.axis_index("c")
        base = wid * bpw
        pltpu.sync_copy(idx_hbm.at[pl.ds(base, bpw)], idx_v)

        rows = (rows0, rows1)
        gsems = (gsem0, gsem1)
        osems = (osem0, osem1)

        def gather_start(c, buf):
            off = pl.multiple_of(c * CHUNK, CHUNK)
            pltpu.async_copy(
                table_hbm.at[idx_v.at[pl.ds(off, CHUNK)]], rows[buf],
                gsems[buf])

        gather_start(0, 0)
        if n_chunks > 1:
            gather_start(1, 1)

        assert n_chunks % 2 == 0

        def step(p, carry):
            for b in range(2):
                c = p * 2 + b
                off = pl.multiple_of(c * CHUNK, CHUNK)
                pltpu.make_async_copy(
                    table_hbm.at[idx_v.at[pl.ds(off, CHUNK)]], rows[b],
                    gsems[b]).wait()
                pltpu.async_copy(
                    rows[b], out_hbm.at[pl.ds(base + off, CHUNK)], osems[b])

                @pl.when(c + 2 < n_chunks)
                def _():
                    noff = pl.multiple_of((c + 2) * CHUNK, CHUNK)
                    pltpu.make_async_copy(
                        rows[b], out_hbm.at[pl.ds(base + off, CHUNK)],
                        osems[b]).wait()
                    pltpu.async_copy(
                        table_hbm.at[idx_v.at[pl.ds(noff, CHUNK)]], rows[b],
                        gsems[b])

            return carry

        lax.fori_loop(0, n_chunks // 2, step, 0)

        for tail in range(max(n_chunks - 2, 0), n_chunks):
            buf = tail % 2
            off = pl.multiple_of(tail * CHUNK, CHUNK)
            pltpu.make_async_copy(
                rows[buf], out_hbm.at[pl.ds(base + off, CHUNK)],
                osems[buf]).wait()

    return gather_kernel


def kernel(x, table):
    B0, S = x.shape
    V, D = table.shape
    B = B0 * S
    flat_idx = x.reshape(B).astype(jnp.int32)
    out = _build_gather(B, V, D)(table, flat_idx)
    return out.reshape(B0, S, D)
